# Initial kernel scaffold; baseline (speedup 1.0000x reference)
import jax
import jax.numpy as jnp
from jax.experimental import pallas as pl


def kernel(x, emb):
    raise NotImplementedError("write your pallas kernel here")


# speedup vs baseline: 2.7531x; 2.7531x over previous
import jax
import jax.numpy as jnp
from jax.experimental import pallas as pl

_DIM = 1024
_SCALE = _DIM ** (-0.5)
_BLOCK_ROWS = 512


def _scale_copy_body(emb_ref, o_ref):
    o_ref[...] = emb_ref[...] * _SCALE


def kernel(x, emb):
    seq_len = x.shape[1]
    emb_used = emb[:seq_len]
    grid = (seq_len // _BLOCK_ROWS,)
    return pl.pallas_call(
        _scale_copy_body,
        grid=grid,
        in_specs=[pl.BlockSpec((_BLOCK_ROWS, _DIM), lambda i: (i, 0))],
        out_specs=pl.BlockSpec((_BLOCK_ROWS, _DIM), lambda i: (i, 0)),
        out_shape=jax.ShapeDtypeStruct((seq_len, _DIM), emb.dtype),
    )(emb_used)


# speedup vs baseline: 3.0192x; 1.0966x over previous
import jax
import jax.numpy as jnp
from jax.experimental import pallas as pl

_DIM = 1024
_SCALE = _DIM ** (-0.5)
_BLOCK_ROWS = 1024


def _scale_copy_body(emb_ref, o_ref):
    o_ref[...] = emb_ref[...] * _SCALE


def kernel(x, emb):
    seq_len = x.shape[1]
    emb_used = emb[:seq_len]
    grid = (seq_len // _BLOCK_ROWS,)
    return pl.pallas_call(
        _scale_copy_body,
        grid=grid,
        in_specs=[pl.BlockSpec((_BLOCK_ROWS, _DIM), lambda i: (i, 0))],
        out_specs=pl.BlockSpec((_BLOCK_ROWS, _DIM), lambda i: (i, 0)),
        out_shape=jax.ShapeDtypeStruct((seq_len, _DIM), emb.dtype),
    )(emb_used)


# speedup vs baseline: 3.2434x; 1.0742x over previous
import jax
import jax.numpy as jnp
from jax.experimental import pallas as pl

_DIM = 1024
_SCALE = _DIM ** (-0.5)
_BLOCK_ROWS = 2048


def _scale_copy_body(emb_ref, o_ref):
    o_ref[...] = emb_ref[...] * _SCALE


def kernel(x, emb):
    seq_len = x.shape[1]
    emb_used = emb[:seq_len]
    grid = (seq_len // _BLOCK_ROWS,)
    return pl.pallas_call(
        _scale_copy_body,
        grid=grid,
        in_specs=[pl.BlockSpec((_BLOCK_ROWS, _DIM), lambda i: (i, 0))],
        out_specs=pl.BlockSpec((_BLOCK_ROWS, _DIM), lambda i: (i, 0)),
        out_shape=jax.ShapeDtypeStruct((seq_len, _DIM), emb.dtype),
    )(emb_used)
